# baseline (speedup 1.0000x reference)
import jax
import jax.numpy as jnp
from jax.experimental import pallas as pl
from jax.experimental.pallas import tpu as pltpu

_EPS = 1e-6


def _gem_body(p_ref, x_ref, o_ref, *, hw: int):
    p = p_ref[0]
    x = x_ref[...]
    tile_n = o_ref.shape[0]
    inv_p = 1.0 / p
    for j in range(0, tile_n, 8):
        acc = jnp.zeros((min(8, tile_n - j), o_ref.shape[1]), jnp.float32)
        for h in range(hw):
            xc = jnp.maximum(x[h, j:j + 8], jnp.float32(_EPS))
            acc = acc + jnp.exp(p * jnp.log(xc))
        m = acc * jnp.float32(1.0 / hw)
        o_ref[j:j + 8, :] = jnp.exp(jnp.log(m) * inv_p).astype(o_ref.dtype)


def kernel(x: jax.Array, p: jax.Array) -> jax.Array:
    N, C, H, W = x.shape
    HW = H * W
    xt = jnp.transpose(x, (2, 3, 0, 1)).reshape(HW, N, C)

    tile_n = N
    for cand in (32, 16, 8, 4, 2):
        if N % cand == 0 and N // cand >= 4:
            tile_n = cand
            break

    out2d = pl.pallas_call(
        lambda pr, xr, orr: _gem_body(pr, xr, orr, hw=HW),
        out_shape=jax.ShapeDtypeStruct((N, C), x.dtype),
        grid=(N // tile_n,),
        in_specs=[
            pl.BlockSpec(memory_space=pltpu.MemorySpace.SMEM),
            pl.BlockSpec((HW, tile_n, C), lambda i: (0, i, 0)),
        ],
        out_specs=pl.BlockSpec((tile_n, C), lambda i: (i, 0)),
        compiler_params=pltpu.CompilerParams(
            dimension_semantics=("parallel",),
            vmem_limit_bytes=48 * 1024 * 1024,
        ),
    )(p, xt)

    return out2d.reshape(N, C, 1, 1)
